# baseline (speedup 1.0000x reference)
import functools

import jax
import jax.numpy as jnp
from jax import lax
from jax.experimental import pallas as pl
from jax.experimental.pallas import tpu as pltpu
from jax.experimental.pallas import tpu_sc as plsc

B = 4096
L = 16
D = 300
DP = 384
O = 2048

NC = 2
NS = 16
NW = NC * NS

ROWS = B * L
R_PER_W = ROWS // NW
CH = 128
N_CHUNKS = R_PER_W // CH

BB = 256
NB = B // BB


def _sc_gather(idx, table):
    mesh = plsc.VectorSubcoreMesh(core_axis_name="c", subcore_axis_name="s")

    @functools.partial(
        pl.kernel,
        out_type=jax.ShapeDtypeStruct((ROWS, DP), jnp.float32),
        mesh=mesh,
        scratch_types=[
            pltpu.VMEM((R_PER_W,), jnp.int32),
            pltpu.VMEM((CH, DP), jnp.float32),
            pltpu.VMEM((CH, DP), jnp.float32),
            pltpu.SemaphoreType.DMA,
            pltpu.SemaphoreType.DMA,
            pltpu.SemaphoreType.DMA,
            pltpu.SemaphoreType.DMA,
        ],
    )
    def k(idx_hbm, table_hbm, out_hbm, idx_v, rows0, rows1, g0, g1, o0, o1):
        wid = lax.axis_index("s") * NC + lax.axis_index("c")
        base = wid * R_PER_W
        bufs = (rows0, rows1)
        gsems = (g0, g1)
        osems = (o0, o1)

        pltpu.sync_copy(idx_hbm.at[pl.ds(base, R_PER_W)], idx_v)

        def gather(c):
            return pltpu.async_copy(
                table_hbm.at[idx_v.at[pl.ds(c * CH, CH)]],
                bufs[c % 2],
                gsems[c % 2],
            )

        ghs = [gather(0)]
        ohs = [None] * N_CHUNKS
        for c in range(N_CHUNKS):
            ghs[c].wait()
            if c + 1 < N_CHUNKS:
                if c >= 1:
                    ohs[c - 1].wait()
                ghs.append(gather(c + 1))
            ohs[c] = pltpu.async_copy(
                bufs[c % 2], out_hbm.at[pl.ds(base + c * CH, CH)], osems[c % 2]
            )
        ohs[N_CHUNKS - 2].wait()
        ohs[N_CHUNKS - 1].wait()

    return k(idx, table)


PAD_BLK = 4096


def _pad_body(t_ref, out_ref):
    out_ref[...] = jnp.concatenate(
        [t_ref[...], jnp.zeros((t_ref.shape[0], DP - D), jnp.float32)], axis=1
    )


def _tc_pad(table):
    v = table.shape[0]
    grid = ((v + PAD_BLK - 1) // PAD_BLK,)
    return pl.pallas_call(
        _pad_body,
        grid=grid,
        in_specs=[pl.BlockSpec((PAD_BLK, D), lambda i: (i, 0))],
        out_specs=pl.BlockSpec((PAD_BLK, DP), lambda i: (i, 0)),
        out_shape=jax.ShapeDtypeStruct((v, DP), jnp.float32),
    )(table)


def _tc_body(emb_ref, wt_ref, b_ref, out_ref):
    h = jnp.dot(
        emb_ref[...].astype(jnp.bfloat16),
        wt_ref[...],
        preferred_element_type=jnp.float32,
    )
    h = jnp.maximum(h + b_ref[...], 0.0)
    out_ref[...] = jnp.max(h.reshape(BB, L, O), axis=1)


def _tc_fused(emb, wt, b2):
    return pl.pallas_call(
        _tc_body,
        grid=(NB,),
        in_specs=[
            pl.BlockSpec((BB * L, DP), lambda i: (i, 0)),
            pl.BlockSpec((DP, O), lambda i: (0, 0)),
            pl.BlockSpec((1, O), lambda i: (0, 0)),
        ],
        out_specs=pl.BlockSpec((BB, O), lambda i: (i, 0)),
        out_shape=jax.ShapeDtypeStruct((B, O), jnp.float32),
        compiler_params=pltpu.CompilerParams(
            dimension_semantics=("parallel",),
        ),
    )(emb, wt, b2)


def kernel(x, table, W, b):
    idx = x.astype(jnp.int32).reshape(ROWS)
    table_p = _tc_pad(table)
    emb = _sc_gather(idx, table_p)
    wt = jnp.pad(W.T, ((0, DP - D), (0, 0))).astype(jnp.bfloat16)
    b2 = b.reshape(1, O)
    return _tc_fused(emb, wt, b2)


# speedup vs baseline: 1.2674x; 1.2674x over previous
import functools

import jax
import jax.numpy as jnp
from jax import lax
from jax.experimental import pallas as pl
from jax.experimental.pallas import tpu as pltpu
from jax.experimental.pallas import tpu_sc as plsc

B = 4096
L = 16
D = 300
DP = 384
O = 2048

NC = 2
NS = 16
NW = NC * NS

ROWS = B * L
R_PER_W = ROWS // NW
CH = 128
N_CHUNKS = R_PER_W // CH

BB = 128
NB = B // BB


def _sc_gather(idx, table):
    mesh = plsc.VectorSubcoreMesh(core_axis_name="c", subcore_axis_name="s")

    @functools.partial(
        pl.kernel,
        out_type=jax.ShapeDtypeStruct((ROWS, DP), jnp.float32),
        mesh=mesh,
        scratch_types=[
            pltpu.VMEM((R_PER_W,), jnp.int32),
            pltpu.VMEM((CH, DP), jnp.float32),
            pltpu.VMEM((CH, DP), jnp.float32),
            pltpu.SemaphoreType.DMA,
            pltpu.SemaphoreType.DMA,
            pltpu.SemaphoreType.DMA,
            pltpu.SemaphoreType.DMA,
        ],
    )
    def k(idx_hbm, table_hbm, out_hbm, idx_v, rows0, rows1, g0, g1, o0, o1):
        wid = lax.axis_index("s") * NC + lax.axis_index("c")
        base = wid * R_PER_W
        bufs = (rows0, rows1)
        gsems = (g0, g1)
        osems = (o0, o1)

        pltpu.sync_copy(idx_hbm.at[pl.ds(base, R_PER_W)], idx_v)

        def gather(c):
            return pltpu.async_copy(
                table_hbm.at[idx_v.at[pl.ds(c * CH, CH)]],
                bufs[c % 2],
                gsems[c % 2],
            )

        ghs = [gather(0)]
        ohs = [None] * N_CHUNKS
        for c in range(N_CHUNKS):
            ghs[c].wait()
            if c + 1 < N_CHUNKS:
                if c >= 1:
                    ohs[c - 1].wait()
                ghs.append(gather(c + 1))
            ohs[c] = pltpu.async_copy(
                bufs[c % 2], out_hbm.at[pl.ds(base + c * CH, CH)], osems[c % 2]
            )
        ohs[N_CHUNKS - 2].wait()
        ohs[N_CHUNKS - 1].wait()

    return k(idx, table)


PAD_BLK = 4096


def _pad_body(t_ref, out_ref):
    blk = out_ref.shape[0]
    out_ref[...] = jnp.concatenate(
        [t_ref[...].T, jnp.zeros((blk, DP - D), jnp.float32)], axis=1
    )


def _tc_pad(table_t):
    v = table_t.shape[1]
    grid = ((v + PAD_BLK - 1) // PAD_BLK,)
    return pl.pallas_call(
        _pad_body,
        grid=grid,
        in_specs=[pl.BlockSpec((D, PAD_BLK), lambda i: (0, i))],
        out_specs=pl.BlockSpec((PAD_BLK, DP), lambda i: (i, 0)),
        out_shape=jax.ShapeDtypeStruct((v, DP), jnp.float32),
    )(table_t)


def _tc_body(emb_ref, wt_ref, b_ref, out_ref):
    h = jnp.dot(
        emb_ref[...].astype(jnp.bfloat16),
        wt_ref[...],
        preferred_element_type=jnp.float32,
    )
    h = jnp.maximum(h + b_ref[...], 0.0)
    out_ref[...] = jnp.max(h.reshape(BB, L, O), axis=1)


def _tc_fused(emb, wt, b2):
    return pl.pallas_call(
        _tc_body,
        grid=(NB,),
        in_specs=[
            pl.BlockSpec((BB * L, DP), lambda i: (i, 0)),
            pl.BlockSpec((DP, O), lambda i: (0, 0)),
            pl.BlockSpec((1, O), lambda i: (0, 0)),
        ],
        out_specs=pl.BlockSpec((BB, O), lambda i: (i, 0)),
        out_shape=jax.ShapeDtypeStruct((B, O), jnp.float32),
        compiler_params=pltpu.CompilerParams(
            dimension_semantics=("parallel",),
        ),
    )(emb, wt, b2)


def kernel(x, table, W, b):
    idx = x.astype(jnp.int32).reshape(ROWS)
    table_p = _tc_pad(table.T)
    emb = _sc_gather(idx, table_p)
    wt = jnp.pad(W.T, ((0, DP - D), (0, 0))).astype(jnp.bfloat16)
    b2 = b.reshape(1, O)
    return _tc_fused(emb, wt, b2)
